# Initial kernel scaffold; baseline (speedup 1.0000x reference)
import jax
import jax.numpy as jnp
from jax.experimental import pallas as pl


def kernel(unscaled_seq, scaled_seq, tables):
    raise NotImplementedError("write your pallas kernel here")


# speedup vs baseline: 1.9240x; 1.9240x over previous
import jax
import jax.numpy as jnp
from jax import lax
from jax.experimental import pallas as pl
from jax.experimental.pallas import tpu as pltpu
from jax.experimental.pallas import tpu_sc as plsc

B = 1024
T = 50
N_FEATURES = 100
N_CAT = 26
VOCAB = 100000
EMB = 16
N_CONT = N_FEATURES - N_CAT
INPUT_LEN = N_CAT * EMB + N_CONT

ROWS = B * T
NW = 32
RB = 80
NB = ROWS // RB
BPW = NB // NW
CONT_CHUNK = 5


def _sc_body(idx_hbm, cont_hbm, tbl_hbm, out_hbm, idx_v, emb_v, cont_v, gsem):
    wid = lax.axis_index("s") * 2 + lax.axis_index("c")
    blk0 = wid * BPW

    def field_body(f, carry):
        pltpu.sync_copy(idx_hbm.at[f, pl.ds(blk0, BPW)], idx_v)
        off = f * VOCAB

        def add_body(i, c):
            r = i // (RB // 16)
            j = (i % (RB // 16)) * 16
            idx_v[r, pl.ds(j, 16)] = idx_v[r, pl.ds(j, 16)] + off
            return c

        lax.fori_loop(0, BPW * (RB // 16), add_body, 0)

        copies = [
            pltpu.async_copy(tbl_hbm.at[idx_v.at[b]], emb_v.at[b], gsem)
            for b in range(BPW)
        ]
        for c in copies:
            c.wait()

        pltpu.sync_copy(
            emb_v, out_hbm.at[pl.ds(blk0, BPW), :, pl.ds(f * EMB, EMB)]
        )
        return carry

    lax.fori_loop(0, N_CAT, field_body, 0)

    def cont_body(c, carry):
        base = blk0 + c * CONT_CHUNK
        pltpu.sync_copy(cont_hbm.at[pl.ds(base, CONT_CHUNK)], cont_v)
        pltpu.sync_copy(
            cont_v, out_hbm.at[pl.ds(base, CONT_CHUNK), :, pl.ds(N_CAT * EMB, N_CONT)]
        )
        return carry

    lax.fori_loop(0, BPW // CONT_CHUNK, cont_body, 0)


def kernel(unscaled_seq, scaled_seq, tables):
    idx = unscaled_seq[:, :, :N_CAT].reshape(ROWS, N_CAT)
    idx_t = jnp.transpose(idx).reshape(N_CAT, NB, RB)
    cont = scaled_seq[:, :, N_CAT:].reshape(NB, RB, N_CONT)
    tbl = tables.reshape(N_CAT * VOCAB, EMB)

    mesh = plsc.VectorSubcoreMesh(core_axis_name="c", subcore_axis_name="s")
    out = pl.kernel(
        _sc_body,
        out_type=jax.ShapeDtypeStruct((NB, RB, INPUT_LEN), jnp.float32),
        mesh=mesh,
        compiler_params=pltpu.CompilerParams(use_tc_tiling_on_sc=False),
        scratch_types=[
            pltpu.VMEM((BPW, RB), jnp.int32),
            pltpu.VMEM((BPW, RB, EMB), jnp.float32),
            pltpu.VMEM((CONT_CHUNK, RB, N_CONT), jnp.float32),
            pltpu.SemaphoreType.DMA,
        ],
    )(idx_t, cont, tbl)
    return out.reshape(B, T, INPUT_LEN)


# speedup vs baseline: 1.9728x; 1.0254x over previous
import jax
import jax.numpy as jnp
from jax import lax
from jax.experimental import pallas as pl
from jax.experimental.pallas import tpu as pltpu
from jax.experimental.pallas import tpu_sc as plsc

B = 1024
T = 50
N_FEATURES = 100
N_CAT = 26
VOCAB = 100000
EMB = 16
N_CONT = N_FEATURES - N_CAT
EMB_LEN = N_CAT * EMB
INPUT_LEN = EMB_LEN + N_CONT

ROWS = B * T
NW = 32
RB = 80
NB = ROWS // RB
BPW = NB // NW
CONT_CHUNK = 5


def _sc_body(
    idx_hbm, cont_hbm, tbl_hbm, out_hbm,
    idx_v0, idx_v1, emb_v0, emb_v1, cont_v0, cont_v1,
    gsem, osem0, osem1,
):
    wid = lax.axis_index("s") * 2 + lax.axis_index("c")
    blk0 = wid * BPW

    def stage_idx(f, idx_v):
        pltpu.sync_copy(idx_hbm.at[f, pl.ds(blk0, BPW)], idx_v)

    def gather_field(f, idx_v, emb_v):
        def add_body(i, c):
            r = i // (RB // 16)
            j = (i % (RB // 16)) * 16
            idx_v[r, pl.ds(j, 16)] = idx_v[r, pl.ds(j, 16)] + f * VOCAB
            return c

        lax.fori_loop(0, BPW * (RB // 16), add_body, 0)
        copies = [
            pltpu.async_copy(tbl_hbm.at[idx_v.at[b]], emb_v.at[b], gsem)
            for b in range(BPW)
        ]
        return copies

    def flush_field(f, emb_v, osem):
        pltpu.async_copy(
            emb_v, out_hbm.at[pl.ds(blk0, BPW), :, pl.ds(f * EMB, EMB)], osem
        )

    def drain_flush(f, emb_v, osem):
        pltpu.make_async_copy(
            emb_v, out_hbm.at[pl.ds(blk0, BPW), :, pl.ds(f * EMB, EMB)], osem
        ).wait()

    def half(f, idx_cur, emb_cur, osem, idx_nxt, f_nxt, first):
        if not first:
            drain_flush(f - 2, emb_cur, osem)
        copies = gather_field(f, idx_cur, emb_cur)
        stage_idx(f_nxt, idx_nxt)
        for c in copies:
            c.wait()
        flush_field(f, emb_cur, osem)

    stage_idx(0, idx_v0)
    half(0, idx_v0, emb_v0, osem0, idx_v1, 1, True)
    half(1, idx_v1, emb_v1, osem1, idx_v0, 2, True)

    def pair_body(i, c):
        f0 = 2 * i
        half(f0, idx_v0, emb_v0, osem0, idx_v1, f0 + 1, False)
        f_nxt = lax.min(f0 + 2, N_CAT - 1)
        half(f0 + 1, idx_v1, emb_v1, osem1, idx_v0, f_nxt, False)
        return c

    lax.fori_loop(1, N_CAT // 2, pair_body, 0)

    def cont_in(c, cont_v):
        pltpu.sync_copy(cont_hbm.at[pl.ds(blk0 + c * CONT_CHUNK, CONT_CHUNK)], cont_v)

    def cont_out(c, cont_v, osem):
        pltpu.async_copy(
            cont_v,
            out_hbm.at[pl.ds(blk0 + c * CONT_CHUNK, CONT_CHUNK), :, pl.ds(EMB_LEN, N_CONT)],
            osem,
        )

    def cont_drain(c, cont_v, osem):
        pltpu.make_async_copy(
            cont_v,
            out_hbm.at[pl.ds(blk0 + c * CONT_CHUNK, CONT_CHUNK), :, pl.ds(EMB_LEN, N_CONT)],
            osem,
        ).wait()

    cont_in(0, cont_v0)
    drain_flush(N_CAT - 2, emb_v0, osem0)
    cont_out(0, cont_v0, osem0)
    cont_in(1, cont_v1)
    drain_flush(N_CAT - 1, emb_v1, osem1)
    cont_out(1, cont_v1, osem1)
    cont_drain(0, cont_v0, osem0)
    cont_in(2, cont_v0)
    cont_out(2, cont_v0, osem0)
    cont_drain(1, cont_v1, osem1)
    cont_in(3, cont_v1)
    cont_out(3, cont_v1, osem1)
    cont_drain(2, cont_v0, osem0)
    cont_drain(3, cont_v1, osem1)


def kernel(unscaled_seq, scaled_seq, tables):
    idx = unscaled_seq[:, :, :N_CAT].reshape(ROWS, N_CAT)
    idx_t = jnp.transpose(idx).reshape(N_CAT, NB, RB)
    cont = scaled_seq[:, :, N_CAT:].reshape(NB, RB, N_CONT)
    tbl = tables.reshape(N_CAT * VOCAB, EMB)

    mesh = plsc.VectorSubcoreMesh(core_axis_name="c", subcore_axis_name="s")
    out = pl.kernel(
        _sc_body,
        out_type=jax.ShapeDtypeStruct((NB, RB, INPUT_LEN), jnp.float32),
        mesh=mesh,
        compiler_params=pltpu.CompilerParams(use_tc_tiling_on_sc=False),
        scratch_types=[
            pltpu.VMEM((BPW, RB), jnp.int32),
            pltpu.VMEM((BPW, RB), jnp.int32),
            pltpu.VMEM((BPW, RB, EMB), jnp.float32),
            pltpu.VMEM((BPW, RB, EMB), jnp.float32),
            pltpu.VMEM((CONT_CHUNK, RB, N_CONT), jnp.float32),
            pltpu.VMEM((CONT_CHUNK, RB, N_CONT), jnp.float32),
            pltpu.SemaphoreType.DMA,
            pltpu.SemaphoreType.DMA,
            pltpu.SemaphoreType.DMA,
        ],
    )(idx_t, cont, tbl)
    return out.reshape(B, T, INPUT_LEN)
